# baseline (speedup 1.0000x reference)
import functools

import jax
import jax.numpy as jnp
from jax import lax
from jax.experimental import pallas as pl
from jax.experimental.pallas import tpu as pltpu
from jax.experimental.pallas import tpu_sc as plsc

_B, _H, _W = 4, 96, 96
_L = 16
_NC = _W // _L
_NW = 32
_BLOCKS = _NW // _B
_RPB = _H // _BLOCKS
_GROUP = 3
_BIG = 1.0e9
_SENT = 20000


def _edt_body(tgt_hbm, tab_hbm, d2a_hbm, d2b_hbm, tgt_v, g_v, tab_v, out_v):
    cid = lax.axis_index("c")
    sid = lax.axis_index("s")
    wid = cid * (_NW // 2) + sid
    b = wid // _BLOCKS
    r0 = (wid % _BLOCKS) * _RPB

    pltpu.sync_copy(tgt_hbm.at[b], tgt_v)
    pltpu.sync_copy(tab_hbm, tab_v)

    def fwd(y, d):
        out = []
        for c in range(_NC):
            t = tgt_v[y, pl.ds(c * _L, _L)]
            near = jnp.where(t != 0, 0.0, _BIG)
            dc = jnp.minimum(d[c] + 1.0, near)
            g_v[y, pl.ds(c * _L, _L)] = dc
            out.append(dc)
        return tuple(out)

    init = tuple(jnp.full((_L,), _BIG, jnp.float32) for _ in range(_NC))
    lax.fori_loop(0, _H, fwd, init)

    def bwd(i, d):
        y = _H - 1 - i
        out = []
        for c in range(_NC):
            prev = g_v[y, pl.ds(c * _L, _L)]
            dc = jnp.minimum(d[c] + 1.0, prev)
            sq = jnp.minimum(dc * dc, float(_SENT))
            g_v[y, pl.ds(c * _L, _L)] = sq
            out.append(dc)
        return tuple(out)

    lax.fori_loop(0, _H, bwd, init)

    for grp in range(_RPB // _GROUP):
        rows = [r0 + grp * _GROUP + k for k in range(_GROUP)]

        def scan_chunk(ct, accs):
            gks = [g_v[rows[k], pl.ds(ct * _L, _L)] for k in range(_GROUP)]
            new = [list(a) for a in accs]
            for j in range(_L):
                tx = ct * _L + j
                tabs = [tab_v[tx, pl.ds(c * _L, _L)] for c in range(_NC)]
                for k in range(_GROUP):
                    s = gks[k][j]
                    for c in range(_NC):
                        new[k][c] = jnp.minimum(new[k][c], tabs[c] + s)
            return tuple(tuple(r) for r in new)

        acc0 = tuple(
            tuple(jnp.full((_L,), float(_SENT + 9025), jnp.float32)
                  for _ in range(_NC))
            for _ in range(_GROUP))
        accs = lax.fori_loop(0, _NC, scan_chunk, acc0)
        for k in range(_GROUP):
            for c in range(_NC):
                out_v[grp * _GROUP + k, pl.ds(c * _L, _L)] = accs[k][c]

    @pl.when(cid == 0)
    def _():
        pltpu.sync_copy(out_v, d2a_hbm.at[sid])

    @pl.when(cid == 1)
    def _():
        pltpu.sync_copy(out_v, d2b_hbm.at[sid])


_edt_sc = pl.kernel(
    _edt_body,
    out_type=[
        jax.ShapeDtypeStruct((_NW // 2, _RPB, _W), jnp.float32),
        jax.ShapeDtypeStruct((_NW // 2, _RPB, _W), jnp.float32),
    ],
    mesh=plsc.VectorSubcoreMesh(core_axis_name="c", subcore_axis_name="s"),
    scratch_types=[
        pltpu.VMEM((_H, _W), jnp.int32),
        pltpu.VMEM((_H, _W), jnp.float32),
        pltpu.VMEM((_W, _W), jnp.float32),
        pltpu.VMEM((_RPB, _W), jnp.float32),
    ],
)


def _loss_body(logits_ref, targets_ref, d2_ref, out_ref):
    maskf = (targets_ref[...] != 0).astype(jnp.float32)
    dist = jnp.sqrt(d2_ref[...])
    preds = jax.nn.sigmoid(logits_ref[...])
    num1 = jnp.sum(preds * dist, axis=(1, 2))
    den1 = jnp.sum(preds, axis=(1, 2))
    num2 = jnp.sum((1.0 - preds) * maskf, axis=(1, 2))
    den2 = jnp.sum(maskf, axis=(1, 2))
    loss = num1 / den1 + num2 / den2
    out_ref[...] = jnp.reshape(jnp.sum(loss) / _B, (1, 1))


@jax.jit
def _run(logits, targets):
    tgt = targets.astype(jnp.int32)
    xs = jnp.arange(_W, dtype=jnp.float32)
    tab = jnp.square(xs[None, :] - xs[:, None])
    d2a, d2b = _edt_sc(tgt, tab)
    d2 = jnp.concatenate([d2a, d2b], axis=0).reshape(_B, _H, _W)
    out = pl.pallas_call(
        _loss_body,
        out_shape=jax.ShapeDtypeStruct((1, 1), jnp.float32),
    )(logits, tgt, d2)
    return out[0, 0]


def kernel(logits, targets):
    return _run(logits, targets)


# speedup vs baseline: 1.2949x; 1.2949x over previous
import functools

import jax
import jax.numpy as jnp
from jax import lax
from jax.experimental import pallas as pl
from jax.experimental.pallas import tpu as pltpu
from jax.experimental.pallas import tpu_sc as plsc

_B, _H, _W = 4, 96, 96
_L = 16
_NC = _W // _L
_NW = 32
_BLOCKS = _NW // _B
_RPB = _H // _BLOCKS
_GROUP = 3
_BIG = 1.0e9
_SENT = 20000


def _edt_body(tgt_hbm, tab_hbm, d2a_hbm, d2b_hbm, tgt_v, g_v, tab_v, out_v):
    cid = lax.axis_index("c")
    sid = lax.axis_index("s")
    wid = cid * (_NW // 2) + sid
    b = wid // _BLOCKS
    r0 = (wid % _BLOCKS) * _RPB

    pltpu.sync_copy(tgt_hbm.at[b], tgt_v)
    pltpu.sync_copy(tab_hbm, tab_v)

    def fwd(y, d):
        out = []
        for c in range(_NC):
            t = tgt_v[y, pl.ds(c * _L, _L)]
            near = jnp.where(t != 0, 0.0, _BIG)
            dc = jnp.minimum(d[c] + 1.0, near)
            g_v[y, pl.ds(c * _L, _L)] = dc
            out.append(dc)
        return tuple(out)

    init = tuple(jnp.full((_L,), _BIG, jnp.float32) for _ in range(_NC))
    lax.fori_loop(0, _H, fwd, init)

    def bwd(i, d):
        y = _H - 1 - i
        out = []
        for c in range(_NC):
            prev = g_v[y, pl.ds(c * _L, _L)]
            dc = jnp.minimum(d[c] + 1.0, prev)
            sq = jnp.minimum(dc * dc, float(_SENT))
            g_v[y, pl.ds(c * _L, _L)] = sq
            out.append(dc)
        return tuple(out)

    lax.fori_loop(0, _H, bwd, init)

    sent = jnp.full((_L,), float(_SENT + 9025), jnp.float32)
    for r in range(_RPB):
        for c in range(_NC):
            out_v[r, pl.ds(c * _L, _L)] = sent

    def scan_chunk(ct, carry):
        grows = [g_v[r0 + r, pl.ds(ct * _L, _L)] for r in range(_RPB)]
        for j in range(_L):
            tx = ct * _L + j
            tabs = [tab_v[tx, pl.ds(c * _L, _L)] for c in range(_NC)]
            for r in range(_RPB):
                s = grows[r][j]
                for c in range(_NC):
                    sl = pl.ds(c * _L, _L)
                    out_v[r, sl] = jnp.minimum(out_v[r, sl], tabs[c] + s)
        return carry

    lax.fori_loop(0, _NC, scan_chunk, 0)

    @pl.when(cid == 0)
    def _():
        pltpu.sync_copy(out_v, d2a_hbm.at[sid])

    @pl.when(cid == 1)
    def _():
        pltpu.sync_copy(out_v, d2b_hbm.at[sid])


_edt_sc = pl.kernel(
    _edt_body,
    out_type=[
        jax.ShapeDtypeStruct((_NW // 2, _RPB, _W), jnp.float32),
        jax.ShapeDtypeStruct((_NW // 2, _RPB, _W), jnp.float32),
    ],
    mesh=plsc.VectorSubcoreMesh(core_axis_name="c", subcore_axis_name="s"),
    scratch_types=[
        pltpu.VMEM((_H, _W), jnp.int32),
        pltpu.VMEM((_H, _W), jnp.float32),
        pltpu.VMEM((_W, _W), jnp.float32),
        pltpu.VMEM((_RPB, _W), jnp.float32),
    ],
)


def _loss_body(logits_ref, targets_ref, d2_ref, out_ref):
    maskf = (targets_ref[...] != 0).astype(jnp.float32)
    dist = jnp.sqrt(d2_ref[...])
    preds = jax.nn.sigmoid(logits_ref[...])
    num1 = jnp.sum(preds * dist, axis=(1, 2))
    den1 = jnp.sum(preds, axis=(1, 2))
    num2 = jnp.sum((1.0 - preds) * maskf, axis=(1, 2))
    den2 = jnp.sum(maskf, axis=(1, 2))
    loss = num1 / den1 + num2 / den2
    out_ref[...] = jnp.reshape(jnp.sum(loss) / _B, (1, 1))


@jax.jit
def _run(logits, targets):
    tgt = targets.astype(jnp.int32)
    xs = jnp.arange(_W, dtype=jnp.float32)
    tab = jnp.square(xs[None, :] - xs[:, None])
    d2a, d2b = _edt_sc(tgt, tab)
    d2 = jnp.concatenate([d2a, d2b], axis=0).reshape(_B, _H, _W)
    out = pl.pallas_call(
        _loss_body,
        out_shape=jax.ShapeDtypeStruct((1, 1), jnp.float32),
    )(logits, tgt, d2)
    return out[0, 0]


def kernel(logits, targets):
    return _run(logits, targets)


# speedup vs baseline: 1.3492x; 1.0419x over previous
import functools

import jax
import jax.numpy as jnp
from jax import lax
from jax.experimental import pallas as pl
from jax.experimental.pallas import tpu as pltpu
from jax.experimental.pallas import tpu_sc as plsc

_B, _H, _W = 4, 96, 96
_L = 16
_NC = _W // _L
_NW = 32
_BLOCKS = _NW // _B
_RPB = _H // _BLOCKS
_GROUP = 3
_BIG = 1.0e9
_SENT = 20000


def _edt_body(tgt_hbm, tab_hbm, d2a_hbm, d2b_hbm, tgt_v, g_v, tab_v, out_v,
              sem):
    cid = lax.axis_index("c")
    sid = lax.axis_index("s")
    wid = cid * (_NW // 2) + sid
    b = wid // _BLOCKS
    r0 = (wid % _BLOCKS) * _RPB

    tab_dma = pltpu.async_copy(tab_hbm, tab_v, sem)
    pltpu.sync_copy(tgt_hbm.at[b], tgt_v)

    def fwd(y, d):
        out = []
        for c in range(_NC):
            t = tgt_v[y, pl.ds(c * _L, _L)]
            near = jnp.where(t != 0, 0.0, _BIG)
            dc = jnp.minimum(d[c] + 1.0, near)
            g_v[y, pl.ds(c * _L, _L)] = dc
            out.append(dc)
        return tuple(out)

    init = tuple(jnp.full((_L,), _BIG, jnp.float32) for _ in range(_NC))
    lax.fori_loop(0, _H, fwd, init)

    def bwd(i, d):
        y = _H - 1 - i
        out = []
        for c in range(_NC):
            prev = g_v[y, pl.ds(c * _L, _L)]
            dc = jnp.minimum(d[c] + 1.0, prev)
            sq = jnp.minimum(dc * dc, float(_SENT))
            g_v[y, pl.ds(c * _L, _L)] = sq
            out.append(dc)
        return tuple(out)

    lax.fori_loop(0, _H, bwd, init)

    tab_dma.wait()
    sent = jnp.full((_L,), float(_SENT + 9025), jnp.float32)
    for r in range(_RPB):
        for c in range(_NC):
            out_v[r, pl.ds(c * _L, _L)] = sent

    def scan_chunk(ct, carry):
        grows = [g_v[r0 + r, pl.ds(ct * _L, _L)] for r in range(_RPB)]
        for j in range(_L):
            tx = ct * _L + j
            tabs = [tab_v[tx, pl.ds(c * _L, _L)] for c in range(_NC)]
            for r in range(_RPB):
                s = grows[r][j]
                for c in range(_NC):
                    sl = pl.ds(c * _L, _L)
                    out_v[r, sl] = jnp.minimum(out_v[r, sl], tabs[c] + s)
        return carry

    lax.fori_loop(0, _NC, scan_chunk, 0)

    @pl.when(cid == 0)
    def _():
        pltpu.sync_copy(out_v, d2a_hbm.at[sid])

    @pl.when(cid == 1)
    def _():
        pltpu.sync_copy(out_v, d2b_hbm.at[sid])


_edt_sc = pl.kernel(
    _edt_body,
    out_type=[
        jax.ShapeDtypeStruct((_NW // 2, _RPB, _W), jnp.float32),
        jax.ShapeDtypeStruct((_NW // 2, _RPB, _W), jnp.float32),
    ],
    mesh=plsc.VectorSubcoreMesh(core_axis_name="c", subcore_axis_name="s"),
    scratch_types=[
        pltpu.VMEM((_H, _W), jnp.int32),
        pltpu.VMEM((_H, _W), jnp.float32),
        pltpu.VMEM((_W, _W), jnp.float32),
        pltpu.VMEM((_RPB, _W), jnp.float32),
        pltpu.SemaphoreType.DMA,
    ],
)


def _loss_body(logits_ref, targets_ref, d2_ref, out_ref):
    maskf = (targets_ref[...] != 0).astype(jnp.float32)
    dist = jnp.sqrt(d2_ref[...])
    preds = jax.nn.sigmoid(logits_ref[...])
    num1 = jnp.sum(preds * dist, axis=(1, 2))
    den1 = jnp.sum(preds, axis=(1, 2))
    num2 = jnp.sum((1.0 - preds) * maskf, axis=(1, 2))
    den2 = jnp.sum(maskf, axis=(1, 2))
    loss = num1 / den1 + num2 / den2
    out_ref[...] = jnp.reshape(jnp.sum(loss) / _B, (1, 1))


@jax.jit
def _run(logits, targets):
    tgt = targets.astype(jnp.int32)
    xs = jnp.arange(_W, dtype=jnp.float32)
    tab = jnp.square(xs[None, :] - xs[:, None])
    d2a, d2b = _edt_sc(tgt, tab)
    d2 = jnp.concatenate([d2a, d2b], axis=0).reshape(_B, _H, _W)
    out = pl.pallas_call(
        _loss_body,
        out_shape=jax.ShapeDtypeStruct((1, 1), jnp.float32),
    )(logits, tgt, d2)
    return out[0, 0]


def kernel(logits, targets):
    return _run(logits, targets)
